# Initial kernel scaffold; baseline (speedup 1.0000x reference)
import functools

import jax
import jax.numpy as jnp
from jax import lax
from jax.experimental import pallas as pl
from jax.experimental.pallas import tpu as pltpu
from jax.experimental.pallas import tpu_sc as plsc


def _sc_gather(word_table, idx_flat):
    BS = idx_flat.shape[0]
    D = word_table.shape[1]
    info = plsc.get_sparse_core_info()
    NW = info.num_cores * info.num_subcores
    per_w = BS // NW
    C = 64
    n_chunks = per_w // C

    mesh = plsc.VectorSubcoreMesh(core_axis_name="c", subcore_axis_name="s")

    @functools.partial(
        pl.kernel,
        mesh=mesh,
        out_type=jax.ShapeDtypeStruct((BS, D), jnp.float32),
        scratch_types=[
            pltpu.VMEM((C,), jnp.int32),
            pltpu.VMEM((C, D), jnp.float32),
            pltpu.SemaphoreType.DMA,
        ],
    )
    def gather_kernel(table_hbm, idx_hbm, out_hbm, idx_v, rows_v, sem):
        wid = lax.axis_index("s") * info.num_cores + lax.axis_index("c")
        base = wid * per_w

        def body(i, carry):
            off = base + i * C
            pltpu.sync_copy(idx_hbm.at[pl.ds(off, C)], idx_v)
            pltpu.async_copy(table_hbm.at[idx_v], rows_v, sem).wait()
            pltpu.sync_copy(rows_v, out_hbm.at[pl.ds(off, C)])
            return carry

        lax.fori_loop(0, n_chunks, body, 0)

    return gather_kernel(word_table, idx_flat)


def _tc_add_ln(word_emb, tt3, pos_table, consts, S, eps=1e-5):
    BS, D = word_emb.shape
    SBLK = 256
    n_blocks = BS // SBLK
    s_blocks = S // SBLK

    def body(tt_ref, emb_ref, pos_ref, const_ref, out_ref):
        x = emb_ref[...] + pos_ref[...]
        tt = tt_ref[0, 0, :]
        t0 = const_ref[2, :][None, :]
        t1 = const_ref[3, :][None, :]
        x = x + jnp.where((tt == 1)[:, None], t1, t0)
        mu = jnp.mean(x, axis=-1, keepdims=True)
        xc = x - mu
        var = jnp.mean(xc * xc, axis=-1, keepdims=True)
        y = xc * lax.rsqrt(var + eps)
        out_ref[...] = y * const_ref[0, :][None, :] + const_ref[1, :][None, :]

    return pl.pallas_call(
        body,
        grid=(n_blocks,),
        in_specs=[
            pl.BlockSpec((1, 1, SBLK), lambda i: (i, 0, 0)),
            pl.BlockSpec((SBLK, D), lambda i: (i, 0)),
            pl.BlockSpec((SBLK, D), lambda i: (i % s_blocks, 0)),
            pl.BlockSpec((8, D), lambda i: (0, 0)),
        ],
        out_specs=pl.BlockSpec((SBLK, D), lambda i: (i, 0)),
        out_shape=jax.ShapeDtypeStruct((BS, D), jnp.float32),
    )(tt3, word_emb, pos_table, consts)


def kernel(input_ids, token_type_ids, word_table, pos_table, type_table,
           ln_gamma, ln_beta):
    B, S = input_ids.shape
    D = word_table.shape[1]
    BS = B * S

    idx_flat = input_ids.reshape(BS)
    word_emb = _sc_gather(word_table, idx_flat)

    consts = jnp.concatenate(
        [ln_gamma[None, :], ln_beta[None, :], type_table,
         jnp.zeros((4, D), jnp.float32)], axis=0)

    SBLK = 256
    tt3 = token_type_ids.reshape(BS // SBLK, 1, SBLK)

    out = _tc_add_ln(word_emb, tt3, pos_table, consts, S)
    return out.reshape(B, S, D)


# speedup vs baseline: 1.6238x; 1.6238x over previous
import functools

import jax
import jax.numpy as jnp
from jax import lax
from jax.experimental import pallas as pl
from jax.experimental.pallas import tpu as pltpu
from jax.experimental.pallas import tpu_sc as plsc


def _sc_gather(word_table, idx_flat):
    BS = idx_flat.shape[0]
    D = word_table.shape[1]
    info = plsc.get_sparse_core_info()
    NW = info.num_cores * info.num_subcores
    per_w = BS // NW
    C = 64
    n_chunks = per_w // C

    mesh = plsc.VectorSubcoreMesh(core_axis_name="c", subcore_axis_name="s")

    @functools.partial(
        pl.kernel,
        mesh=mesh,
        out_type=jax.ShapeDtypeStruct((BS, D), jnp.float32),
        scratch_types=[
            pltpu.VMEM((C,), jnp.int32),
            pltpu.VMEM((C, D), jnp.float32),
            pltpu.SemaphoreType.DMA,
        ],
    )
    def gather_kernel(table_hbm, idx_hbm, out_hbm, idx_v, rows_v, sem):
        wid = lax.axis_index("s") * info.num_cores + lax.axis_index("c")
        base = wid * per_w

        def body(i, carry):
            off = base + i * C
            pltpu.sync_copy(idx_hbm.at[pl.ds(off, C)], idx_v)
            pltpu.async_copy(table_hbm.at[idx_v], rows_v, sem).wait()
            pltpu.sync_copy(rows_v, out_hbm.at[pl.ds(off, C)])
            return carry

        lax.fori_loop(0, n_chunks, body, 0)

    return gather_kernel(word_table, idx_flat)


def _tc_add_ln(word_emb, tt2, pos_table, consts, S, eps=1e-5):
    BS, D = word_emb.shape
    SBLK = 256
    n_blocks = BS // SBLK
    s_blocks = S // SBLK

    def body(tt_ref, emb_ref, pos_ref, const_ref, out_ref):
        x = emb_ref[...] + pos_ref[...]
        tt = tt_ref[...]
        t0 = const_ref[2, :][None, :]
        t1 = const_ref[3, :][None, :]
        x = x + jnp.where(tt == 1, t1, t0)
        mu = jnp.mean(x, axis=-1, keepdims=True)
        xc = x - mu
        var = jnp.mean(xc * xc, axis=-1, keepdims=True)
        y = xc * lax.rsqrt(var + eps)
        out_ref[...] = y * const_ref[0, :][None, :] + const_ref[1, :][None, :]

    return pl.pallas_call(
        body,
        grid=(n_blocks,),
        in_specs=[
            pl.BlockSpec((SBLK, 1), lambda i: (i, 0)),
            pl.BlockSpec((SBLK, D), lambda i: (i, 0)),
            pl.BlockSpec((SBLK, D), lambda i: (i % s_blocks, 0)),
            pl.BlockSpec((8, D), lambda i: (0, 0)),
        ],
        out_specs=pl.BlockSpec((SBLK, D), lambda i: (i, 0)),
        out_shape=jax.ShapeDtypeStruct((BS, D), jnp.float32),
    )(tt2, word_emb, pos_table, consts)


def kernel(input_ids, token_type_ids, word_table, pos_table, type_table,
           ln_gamma, ln_beta):
    B, S = input_ids.shape
    D = word_table.shape[1]
    BS = B * S

    idx_flat = input_ids.reshape(BS)
    word_emb = _sc_gather(word_table, idx_flat)

    consts = jnp.concatenate(
        [ln_gamma[None, :], ln_beta[None, :], type_table,
         jnp.zeros((4, D), jnp.float32)], axis=0)

    tt2 = token_type_ids.reshape(BS, 1)

    out = _tc_add_ln(word_emb, tt2, pos_table, consts, S)
    return out.reshape(B, S, D)


# speedup vs baseline: 1.9450x; 1.1978x over previous
import functools

import jax
import jax.numpy as jnp
from jax import lax
from jax.experimental import pallas as pl
from jax.experimental.pallas import tpu as pltpu
from jax.experimental.pallas import tpu_sc as plsc


def _sc_gather(word_table, idx_flat):
    BS = idx_flat.shape[0]
    D = word_table.shape[1]
    info = plsc.get_sparse_core_info()
    NW = info.num_cores * info.num_subcores
    per_w = BS // NW
    C = 32
    n_chunks = per_w // C

    mesh = plsc.VectorSubcoreMesh(core_axis_name="c", subcore_axis_name="s")

    @functools.partial(
        pl.kernel,
        mesh=mesh,
        out_type=jax.ShapeDtypeStruct((BS, D), jnp.float32),
        scratch_types=[
            pltpu.VMEM((per_w,), jnp.int32),
            pltpu.VMEM((C, D), jnp.float32),
            pltpu.VMEM((C, D), jnp.float32),
            pltpu.SemaphoreType.DMA,
            pltpu.SemaphoreType.DMA,
        ],
    )
    def gather_kernel(table_hbm, idx_hbm, out_hbm, idx_v, buf0, buf1, sem0, sem1):
        wid = lax.axis_index("s") * info.num_cores + lax.axis_index("c")
        base = wid * per_w
        bufs = (buf0, buf1)
        sems = (sem0, sem1)

        pltpu.sync_copy(idx_hbm.at[pl.ds(base, per_w)], idx_v)

        def start(i):
            return pltpu.async_copy(
                word_table_at(i), bufs[i % 2], sems[i % 2])

        def word_table_at(i):
            return table_hbm.at[idx_v.at[pl.ds(i * C, C)]]

        copies = [None] * n_chunks
        copies[0] = start(0)
        for i in range(n_chunks):
            if i + 1 < n_chunks:
                copies[i + 1] = start(i + 1)
            copies[i].wait()
            pltpu.sync_copy(bufs[i % 2], out_hbm.at[pl.ds(base + i * C, C)])

    return gather_kernel(word_table, idx_flat)


def _tc_add_ln(word_emb, tt3, pos_table, consts, eps=1e-5):
    B, S, D = word_emb.shape
    SBLK = 512
    s_blocks = S // SBLK

    def body(tt_ref, emb_ref, pos_ref, const_ref, out_ref):
        x = emb_ref[0] + pos_ref[...]
        tt = tt_ref[0]
        t0 = const_ref[2, :][None, :]
        t1 = const_ref[3, :][None, :]
        x = x + jnp.where(tt == 1, t1, t0)
        mu = jnp.mean(x, axis=-1, keepdims=True)
        xc = x - mu
        var = jnp.mean(xc * xc, axis=-1, keepdims=True)
        y = xc * lax.rsqrt(var + eps)
        out_ref[0] = y * const_ref[0, :][None, :] + const_ref[1, :][None, :]

    return pl.pallas_call(
        body,
        grid=(s_blocks, B),
        in_specs=[
            pl.BlockSpec((1, SBLK, 1), lambda i, j: (j, i, 0)),
            pl.BlockSpec((1, SBLK, D), lambda i, j: (j, i, 0)),
            pl.BlockSpec((SBLK, D), lambda i, j: (i, 0)),
            pl.BlockSpec((8, D), lambda i, j: (0, 0)),
        ],
        out_specs=pl.BlockSpec((1, SBLK, D), lambda i, j: (j, i, 0)),
        out_shape=jax.ShapeDtypeStruct((B, S, D), jnp.float32),
    )(tt3, word_emb, pos_table, consts)


def kernel(input_ids, token_type_ids, word_table, pos_table, type_table,
           ln_gamma, ln_beta):
    B, S = input_ids.shape
    D = word_table.shape[1]
    BS = B * S

    idx_flat = input_ids.reshape(BS)
    word_emb = _sc_gather(word_table, idx_flat).reshape(B, S, D)

    consts = jnp.concatenate(
        [ln_gamma[None, :], ln_beta[None, :], type_table,
         jnp.zeros((4, D), jnp.float32)], axis=0)

    tt3 = token_type_ids.reshape(B, S, 1)

    return _tc_add_ln(word_emb, tt3, pos_table, consts)
